# baseline (speedup 1.0000x reference)
import functools

import jax
import jax.numpy as jnp
from jax import lax
from jax.experimental import pallas as pl
from jax.experimental.pallas import tpu as pltpu
from jax.experimental.pallas import tpu_sc as plsc

B = 16384
E = 32
_RPS = 4
_W = _RPS * E
_CH = 256


def _make_sc_gather():
    info = plsc.get_sparse_core_info()
    NC, NS = info.num_cores, info.num_subcores
    NW = NC * NS
    b_per_w = B // NW
    n_ch = b_per_w // _CH
    mesh = plsc.VectorSubcoreMesh(core_axis_name="c", subcore_axis_name="s")

    @functools.partial(
        pl.kernel,
        mesh=mesh,
        out_type=[
            jax.ShapeDtypeStruct((B, _W), jnp.float32),
            jax.ShapeDtypeStruct((B, _W), jnp.float32),
        ],
        scratch_types=[
            pltpu.VMEM((b_per_w,), jnp.int32),
            pltpu.VMEM((b_per_w,), jnp.int32),
            pltpu.VMEM((_CH, _W), jnp.float32),
            pltpu.VMEM((_CH, _W), jnp.float32),
            pltpu.SemaphoreType.DMA,
            pltpu.SemaphoreType.DMA,
            pltpu.SemaphoreType.DMA,
        ],
    )
    def sc_gather(user_t4, item_t4, user_sidx, item_sidx,
                  uout, iout, uidx_v, iidx_v, ubuf, ibuf, usem, isem, osem):
        wid = lax.axis_index("s") * NC + lax.axis_index("c")
        base = wid * b_per_w
        pltpu.sync_copy(user_sidx.at[pl.ds(base, b_per_w)], uidx_v)
        pltpu.sync_copy(item_sidx.at[pl.ds(base, b_per_w)], iidx_v)

        def chunk(c, _):
            off = c * _CH
            cu = pltpu.async_copy(
                user_t4.at[uidx_v.at[pl.ds(off, _CH)]], ubuf, usem)
            ci = pltpu.async_copy(
                item_t4.at[iidx_v.at[pl.ds(off, _CH)]], ibuf, isem)
            cu.wait()
            co_u = pltpu.async_copy(ubuf, uout.at[pl.ds(base + off, _CH)],
                                    osem)
            ci.wait()
            co_i = pltpu.async_copy(ibuf, iout.at[pl.ds(base + off, _CH)],
                                    osem)
            co_u.wait()
            co_i.wait()
            return _

        lax.fori_loop(0, n_ch, chunk, 0)

    return sc_gather


_sc_gather = _make_sc_gather()


_BB = 2048


def _select_row(x4, rid):
    acc = x4[:, 0:E]
    for r in range(1, _RPS):
        acc = jnp.where(rid == r, x4[:, r * E:(r + 1) * E], acc)
    return acc


def _mlp_body(ue4, ie4, uid, iid, f, wws, w0u, w0i, w0f, b0, w1, b1, w2, b2,
              wod, cb, out):
    ue = _select_row(ue4[...], lax.rem(uid[...], _RPS))
    ie = _select_row(ie4[...], lax.rem(iid[...], _RPS))
    fv = f[...]
    h = (ue @ w0u[...] + ie @ w0i[...] + fv @ w0f[...] + b0[...])
    h = jnp.maximum(h, 0.0)
    h = jnp.maximum(h @ w1[...] + b1[...], 0.0)
    h = jnp.maximum(h @ w2[...] + b2[...], 0.0)
    logit = fv @ wws[...] + h @ wod[...] + cb[...]
    out[...] = jax.nn.sigmoid(logit)


def _mlp(ue4, ie4, uid, iid, features,
         wws, w0u, w0i, w0f, b0, w1, b1, w2, b2, wod, cb):
    n_f = features.shape[1]
    d0, d1, d2 = w0u.shape[1], w1.shape[1], w2.shape[1]
    grid = (B // _BB,)
    row = lambda i: (i, 0)
    zero = lambda i: (0, 0)
    return pl.pallas_call(
        _mlp_body,
        grid=grid,
        in_specs=[
            pl.BlockSpec((_BB, _W), row),
            pl.BlockSpec((_BB, _W), row),
            pl.BlockSpec((_BB, 1), row),
            pl.BlockSpec((_BB, 1), row),
            pl.BlockSpec((_BB, n_f), row),
            pl.BlockSpec((n_f, 1), zero),
            pl.BlockSpec((E, d0), zero),
            pl.BlockSpec((E, d0), zero),
            pl.BlockSpec((n_f, d0), zero),
            pl.BlockSpec((1, d0), zero),
            pl.BlockSpec((d0, d1), zero),
            pl.BlockSpec((1, d1), zero),
            pl.BlockSpec((d1, d2), zero),
            pl.BlockSpec((1, d2), zero),
            pl.BlockSpec((d2, 1), zero),
            pl.BlockSpec((1, 1), zero),
        ],
        out_specs=pl.BlockSpec((_BB, 1), row),
        out_shape=jax.ShapeDtypeStruct((B, 1), jnp.float32),
        compiler_params=pltpu.CompilerParams(
            dimension_semantics=("arbitrary",),
        ),
    )(ue4, ie4, uid, iid, features,
      wws, w0u, w0i, w0f, b0, w1, b1, w2, b2, wod, cb)


def kernel(user_ids, item_ids, features, user_table, item_table,
           W_wide, b_wide, W0, b0, W1, b1, W2, b2, Wo, bo):
    ut4 = user_table.reshape(-1, _W)
    it4 = item_table.reshape(-1, _W)
    user_s = lax.shift_right_logical(user_ids, 2)
    item_s = lax.shift_right_logical(item_ids, 2)
    ue4, ie4 = _sc_gather(ut4, it4, user_s, item_s)

    wo0 = Wo[0, 0]
    wws = W_wide * wo0
    wod = Wo[1:, :]
    cb = (b_wide * wo0 + bo).reshape(1, 1)
    w0u = W0[:E, :]
    w0i = W0[E:2 * E, :]
    w0f = W0[2 * E:, :]

    return _mlp(ue4, ie4, user_ids.reshape(-1, 1), item_ids.reshape(-1, 1),
                features, wws, w0u, w0i, w0f, b0.reshape(1, -1),
                W1, b1.reshape(1, -1), W2, b2.reshape(1, -1), wod, cb)
